# baseline (speedup 1.0000x reference)
import functools

import jax
import jax.numpy as jnp
from jax import lax
from jax.experimental import pallas as pl
from jax.experimental.pallas import tpu as pltpu
from jax.experimental.pallas import tpu_sc as plsc

_B = 16384
_E = 32
_PK = 8
_PW = 128 // _PK
_NC = 2
_NS = 16
_NW = _NC * _NS
_BPW = _B // _NW
_NCH = 8
_CH = _BPW // _NCH

_TC_BLK = 2048


def _pack_table(table):
    tb = table.astype(jnp.bfloat16)
    tp = jax.lax.bitcast_convert_type(
        tb.reshape(table.shape[0], _PW, 2), jnp.float32)
    return tp.reshape(table.shape[0] // _PK, 128)


def _sc_gather_pair(user_packed, item_packed, uhi, ihi):
    mesh = plsc.VectorSubcoreMesh(core_axis_name="c", subcore_axis_name="s")

    @functools.partial(
        pl.kernel,
        mesh=mesh,
        out_type=(
            jax.ShapeDtypeStruct((_B, 128), jnp.float32),
            jax.ShapeDtypeStruct((_B, 128), jnp.float32),
        ),
        scratch_types=[
            pltpu.VMEM((_BPW,), jnp.int32),
            pltpu.VMEM((_BPW, 128), jnp.float32),
            pltpu.SemaphoreType.DMA,
        ],
    )
    def gather_kernel(ut_hbm, it_hbm, ui_hbm, ii_hbm, uo_hbm, io_hbm,
                      idx_v, rows_v, sem):
        wid = lax.axis_index("s") * _NC + lax.axis_index("c")
        base = wid * _BPW

        def one_table(tab_hbm, i_hbm, o_hbm):
            pltpu.sync_copy(i_hbm.at[pl.ds(base, _BPW)], idx_v)
            for ch in range(_NCH):
                pltpu.async_copy(
                    tab_hbm.at[idx_v.at[pl.ds(ch * _CH, _CH)]],
                    rows_v.at[pl.ds(ch * _CH, _CH), :], sem)
            pltpu.make_async_copy(tab_hbm.at[pl.ds(0, _BPW)], rows_v,
                                  sem).wait()
            pltpu.sync_copy(rows_v, o_hbm.at[pl.ds(base, _BPW)])

        one_table(ut_hbm, ui_hbm, uo_hbm)
        one_table(it_hbm, ii_hbm, io_hbm)

    return gather_kernel(user_packed, item_packed, uhi, ihi)


def _select_unpack(g, sub):
    out = jnp.where(sub == 0, g[:, 0 * _PW:1 * _PW], 0.0)
    for q in range(1, _PK):
        out += jnp.where(sub == q, g[:, q * _PW:(q + 1) * _PW], 0.0)
    bits = jax.lax.bitcast_convert_type(out, jnp.int32)
    lo = jax.lax.bitcast_convert_type(bits << 16, jnp.float32)
    hi = jax.lax.bitcast_convert_type(bits & jnp.int32(-65536), jnp.float32)
    return lo, hi


def _tower(x, W0e, W0o, b0, W1, b1, W2, b2):
    def mm(v, W):
        return jnp.dot(v.astype(jnp.bfloat16), W.astype(jnp.bfloat16),
                       preferred_element_type=jnp.float32)
    lo, hi = x
    h = jnp.maximum(mm(lo, W0e) + mm(hi, W0o) + b0, 0.0)
    h = jnp.maximum(mm(h, W1) + b1, 0.0)
    return mm(h, W2) + b2


def _tc_body(ug_ref, ig_ref, us_ref, is_ref,
             uW0er, uW0or, ub0r, uW1r, ub1r, uW2r, ub2r,
             iW0er, iW0or, ib0r, iW1r, ib1r, iW2r, ib2r, o_ref):
    u = _tower(_select_unpack(ug_ref[...], us_ref[...]),
               uW0er[...], uW0or[...], ub0r[...], uW1r[...], ub1r[...],
               uW2r[...], ub2r[...])
    v = _tower(_select_unpack(ig_ref[...], is_ref[...]),
               iW0er[...], iW0or[...], ib0r[...], iW1r[...], ib1r[...],
               iW2r[...], ib2r[...])
    dot = jnp.sum(u * v, axis=1)
    nu = jnp.sqrt(jnp.sum(u * u, axis=1))
    nv = jnp.sqrt(jnp.sum(v * v, axis=1))
    o_ref[...] = dot / (jnp.maximum(nu, 1e-12) * jnp.maximum(nv, 1e-12))


def _tc_towers(u_g, i_g, u_sub, i_sub,
               uW0, ub0, uW1, ub1, uW2, ub2,
               iW0, ib0, iW1, ib1, iW2, ib2):
    def full(a):
        return pl.BlockSpec(a.shape, lambda i: (0,) * a.ndim)

    weights = [uW0[0::2], uW0[1::2], ub0, uW1, ub1, uW2, ub2,
               iW0[0::2], iW0[1::2], ib0, iW1, ib1, iW2, ib2]
    weights = [w.reshape(1, -1) if w.ndim == 1 else w for w in weights]
    return pl.pallas_call(
        _tc_body,
        grid=(_B // _TC_BLK,),
        in_specs=[
            pl.BlockSpec((_TC_BLK, 128), lambda i: (i, 0)),
            pl.BlockSpec((_TC_BLK, 128), lambda i: (i, 0)),
            pl.BlockSpec((_TC_BLK, 1), lambda i: (i, 0)),
            pl.BlockSpec((_TC_BLK, 1), lambda i: (i, 0)),
        ] + [full(w) for w in weights],
        out_specs=pl.BlockSpec((_TC_BLK,), lambda i: (i,)),
        out_shape=jax.ShapeDtypeStruct((_B,), jnp.float32),
    )(u_g, i_g, u_sub, i_sub, *weights)


@jax.jit
def kernel(user_idx, item_idx, user_table, item_table,
           uW0, ub0, uW1, ub1, uW2, ub2,
           iW0, ib0, iW1, ib1, iW2, ib2):
    ui = user_idx.astype(jnp.int32)
    ii = item_idx.astype(jnp.int32)
    up = _pack_table(user_table)
    ip = _pack_table(item_table)
    u_g, i_g = _sc_gather_pair(up, ip, ui >> 3, ii >> 3)
    u_sub = (ui & 7).reshape(_B, 1)
    i_sub = (ii & 7).reshape(_B, 1)
    return _tc_towers(u_g, i_g, u_sub, i_sub,
                      uW0, ub0, uW1, ub1, uW2, ub2,
                      iW0, ib0, iW1, ib1, iW2, ib2)


# speedup vs baseline: 2.1618x; 2.1618x over previous
import functools

import jax
import jax.numpy as jnp
from jax import lax
from jax.experimental import pallas as pl
from jax.experimental.pallas import tpu as pltpu
from jax.experimental.pallas import tpu_sc as plsc

_B = 16384
_E = 32
_NC = 2
_NS = 16
_NW = _NC * _NS
_BPW = _B // _NW
_NCH = 8
_CH = _BPW // _NCH

_TC_BLK = 2048


def _sc_gather_pair(user_packed, item_packed, uhi, ihi):
    mesh = plsc.VectorSubcoreMesh(core_axis_name="c", subcore_axis_name="s")

    @functools.partial(
        pl.kernel,
        mesh=mesh,
        out_type=(
            jax.ShapeDtypeStruct((_B, 128), jnp.float32),
            jax.ShapeDtypeStruct((_B, 128), jnp.float32),
        ),
        scratch_types=[
            pltpu.VMEM((_BPW,), jnp.int32),
            pltpu.VMEM((_BPW, 128), jnp.float32),
            pltpu.SemaphoreType.DMA,
        ],
    )
    def gather_kernel(ut_hbm, it_hbm, ui_hbm, ii_hbm, uo_hbm, io_hbm,
                      idx_v, rows_v, sem):
        wid = lax.axis_index("s") * _NC + lax.axis_index("c")
        base = wid * _BPW

        def one_table(tab_hbm, i_hbm, o_hbm):
            pltpu.sync_copy(i_hbm.at[pl.ds(base, _BPW)], idx_v)
            for ch in range(_NCH):
                pltpu.async_copy(
                    tab_hbm.at[idx_v.at[pl.ds(ch * _CH, _CH)]],
                    rows_v.at[pl.ds(ch * _CH, _CH), :], sem)
            pltpu.make_async_copy(tab_hbm.at[pl.ds(0, _BPW)], rows_v,
                                  sem).wait()
            pltpu.sync_copy(rows_v, o_hbm.at[pl.ds(base, _BPW)])

        one_table(ut_hbm, ui_hbm, uo_hbm)
        one_table(it_hbm, ii_hbm, io_hbm)

    return gather_kernel(user_packed, item_packed, uhi, ihi)


def _select_sub(g, sub):
    out = jnp.where(sub == 0, g[:, 0 * _E:1 * _E], 0.0)
    out += jnp.where(sub == 1, g[:, 1 * _E:2 * _E], 0.0)
    out += jnp.where(sub == 2, g[:, 2 * _E:3 * _E], 0.0)
    out += jnp.where(sub == 3, g[:, 3 * _E:4 * _E], 0.0)
    return out


def _tower(x, W0, b0, W1, b1, W2, b2):
    def mm(v, W):
        return jnp.dot(v.astype(jnp.bfloat16), W.astype(jnp.bfloat16),
                       preferred_element_type=jnp.float32)
    h = jnp.maximum(mm(x, W0) + b0, 0.0)
    h = jnp.maximum(mm(h, W1) + b1, 0.0)
    return mm(h, W2) + b2


def _tc_body(ug_ref, ig_ref, us_ref, is_ref,
             uW0r, ub0r, uW1r, ub1r, uW2r, ub2r,
             iW0r, ib0r, iW1r, ib1r, iW2r, ib2r, o_ref):
    u = _tower(_select_sub(ug_ref[...], us_ref[...]),
               uW0r[...], ub0r[...], uW1r[...], ub1r[...],
               uW2r[...], ub2r[...])
    v = _tower(_select_sub(ig_ref[...], is_ref[...]),
               iW0r[...], ib0r[...], iW1r[...], ib1r[...],
               iW2r[...], ib2r[...])
    dot = jnp.sum(u * v, axis=1)
    nu = jnp.sqrt(jnp.sum(u * u, axis=1))
    nv = jnp.sqrt(jnp.sum(v * v, axis=1))
    o_ref[...] = dot / (jnp.maximum(nu, 1e-12) * jnp.maximum(nv, 1e-12))


def _tc_towers(u_g, i_g, u_sub, i_sub,
               uW0, ub0, uW1, ub1, uW2, ub2,
               iW0, ib0, iW1, ib1, iW2, ib2):
    def full(a):
        return pl.BlockSpec(a.shape, lambda i: (0,) * a.ndim)

    weights = [uW0, ub0, uW1, ub1, uW2, ub2, iW0, ib0, iW1, ib1, iW2, ib2]
    weights = [w.reshape(1, -1) if w.ndim == 1 else w for w in weights]
    return pl.pallas_call(
        _tc_body,
        grid=(_B // _TC_BLK,),
        in_specs=[
            pl.BlockSpec((_TC_BLK, 128), lambda i: (i, 0)),
            pl.BlockSpec((_TC_BLK, 128), lambda i: (i, 0)),
            pl.BlockSpec((_TC_BLK, 1), lambda i: (i, 0)),
            pl.BlockSpec((_TC_BLK, 1), lambda i: (i, 0)),
        ] + [full(w) for w in weights],
        out_specs=pl.BlockSpec((_TC_BLK,), lambda i: (i,)),
        out_shape=jax.ShapeDtypeStruct((_B,), jnp.float32),
    )(u_g, i_g, u_sub, i_sub, *weights)


@jax.jit
def kernel(user_idx, item_idx, user_table, item_table,
           uW0, ub0, uW1, ub1, uW2, ub2,
           iW0, ib0, iW1, ib1, iW2, ib2):
    ui = user_idx.astype(jnp.int32)
    ii = item_idx.astype(jnp.int32)
    up = user_table.reshape(-1, 128)
    ip = item_table.reshape(-1, 128)
    u_g, i_g = _sc_gather_pair(up, ip, ui >> 2, ii >> 2)
    u_sub = (ui & 3).reshape(_B, 1)
    i_sub = (ii & 3).reshape(_B, 1)
    return _tc_towers(u_g, i_g, u_sub, i_sub,
                      uW0, ub0, uW1, ub1, uW2, ub2,
                      iW0, ib0, iW1, ib1, iW2, ib2)


# speedup vs baseline: 3.5850x; 1.6584x over previous
import functools

import jax
import jax.numpy as jnp
from jax import lax
from jax.experimental import pallas as pl
from jax.experimental.pallas import tpu as pltpu
from jax.experimental.pallas import tpu_sc as plsc

_B = 16384
_E = 32
_NC = 2
_NS = 16
_NW = _NC * _NS
_BPW = _B // _NW
_NCH = 8
_CH = _BPW // _NCH

_TC_BLK = 2048
_LB = 32256
_NPB = 32


def _tc_packT(table):
    xt = table.T

    def body(x_ref, o_ref, tmp_ref):
        tmp_ref[...] = x_ref[...].T
        o_ref[...] = jnp.concatenate(
            [tmp_ref[q::4, :] for q in range(4)], axis=1)

    return pl.pallas_call(
        body,
        grid=(_NPB,),
        in_specs=[pl.BlockSpec((_E, _LB), lambda i: (0, i))],
        out_specs=pl.BlockSpec((_LB // 4, 128), lambda i: (i, 0)),
        out_shape=jax.ShapeDtypeStruct((_NPB * (_LB // 4), 128),
                                       jnp.float32),
        scratch_shapes=[pltpu.VMEM((_LB, _E), jnp.float32)],
    )(xt)


def _sc_gather_pair(user_packed, item_packed, uhi, ihi):
    mesh = plsc.VectorSubcoreMesh(core_axis_name="c", subcore_axis_name="s")

    @functools.partial(
        pl.kernel,
        mesh=mesh,
        out_type=(
            jax.ShapeDtypeStruct((_B, 128), jnp.float32),
            jax.ShapeDtypeStruct((_B, 128), jnp.float32),
        ),
        scratch_types=[
            pltpu.VMEM((_BPW,), jnp.int32),
            pltpu.VMEM((_BPW, 128), jnp.float32),
            pltpu.SemaphoreType.DMA,
        ],
    )
    def gather_kernel(ut_hbm, it_hbm, ui_hbm, ii_hbm, uo_hbm, io_hbm,
                      idx_v, rows_v, sem):
        wid = lax.axis_index("s") * _NC + lax.axis_index("c")
        base = wid * _BPW

        def one_table(tab_hbm, i_hbm, o_hbm):
            pltpu.sync_copy(i_hbm.at[pl.ds(base, _BPW)], idx_v)
            for ch in range(_NCH):
                pltpu.async_copy(
                    tab_hbm.at[idx_v.at[pl.ds(ch * _CH, _CH)]],
                    rows_v.at[pl.ds(ch * _CH, _CH), :], sem)
            pltpu.make_async_copy(tab_hbm.at[pl.ds(0, _BPW)], rows_v,
                                  sem).wait()
            pltpu.sync_copy(rows_v, o_hbm.at[pl.ds(base, _BPW)])

        one_table(ut_hbm, ui_hbm, uo_hbm)
        one_table(it_hbm, ii_hbm, io_hbm)

    return gather_kernel(user_packed, item_packed, uhi, ihi)


def _select_sub(g, sub):
    out = jnp.where(sub == 0, g[:, 0 * _E:1 * _E], 0.0)
    out += jnp.where(sub == 1, g[:, 1 * _E:2 * _E], 0.0)
    out += jnp.where(sub == 2, g[:, 2 * _E:3 * _E], 0.0)
    out += jnp.where(sub == 3, g[:, 3 * _E:4 * _E], 0.0)
    return out


def _tower(x, W0, b0, W1, b1, W2, b2):
    def mm(v, W):
        return jnp.dot(v.astype(jnp.bfloat16), W.astype(jnp.bfloat16),
                       preferred_element_type=jnp.float32)
    h = jnp.maximum(mm(x, W0) + b0, 0.0)
    h = jnp.maximum(mm(h, W1) + b1, 0.0)
    return mm(h, W2) + b2


def _tc_body(ug_ref, ig_ref, us_ref, is_ref,
             uW0r, ub0r, uW1r, ub1r, uW2r, ub2r,
             iW0r, ib0r, iW1r, ib1r, iW2r, ib2r, o_ref):
    u = _tower(_select_sub(ug_ref[...], us_ref[...]),
               uW0r[...], ub0r[...], uW1r[...], ub1r[...],
               uW2r[...], ub2r[...])
    v = _tower(_select_sub(ig_ref[...], is_ref[...]),
               iW0r[...], ib0r[...], iW1r[...], ib1r[...],
               iW2r[...], ib2r[...])
    dot = jnp.sum(u * v, axis=1)
    nu = jnp.sqrt(jnp.sum(u * u, axis=1))
    nv = jnp.sqrt(jnp.sum(v * v, axis=1))
    o_ref[...] = dot / (jnp.maximum(nu, 1e-12) * jnp.maximum(nv, 1e-12))


def _tc_towers(u_g, i_g, u_sub, i_sub,
               uW0, ub0, uW1, ub1, uW2, ub2,
               iW0, ib0, iW1, ib1, iW2, ib2):
    def full(a):
        return pl.BlockSpec(a.shape, lambda i: (0,) * a.ndim)

    weights = [uW0, ub0, uW1, ub1, uW2, ub2, iW0, ib0, iW1, ib1, iW2, ib2]
    weights = [w.reshape(1, -1) if w.ndim == 1 else w for w in weights]
    return pl.pallas_call(
        _tc_body,
        grid=(_B // _TC_BLK,),
        in_specs=[
            pl.BlockSpec((_TC_BLK, 128), lambda i: (i, 0)),
            pl.BlockSpec((_TC_BLK, 128), lambda i: (i, 0)),
            pl.BlockSpec((_TC_BLK, 1), lambda i: (i, 0)),
            pl.BlockSpec((_TC_BLK, 1), lambda i: (i, 0)),
        ] + [full(w) for w in weights],
        out_specs=pl.BlockSpec((_TC_BLK,), lambda i: (i,)),
        out_shape=jax.ShapeDtypeStruct((_B,), jnp.float32),
    )(u_g, i_g, u_sub, i_sub, *weights)


@jax.jit
def kernel(user_idx, item_idx, user_table, item_table,
           uW0, ub0, uW1, ub1, uW2, ub2,
           iW0, ib0, iW1, ib1, iW2, ib2):
    ui = user_idx.astype(jnp.int32)
    ii = item_idx.astype(jnp.int32)
    up = _tc_packT(user_table)
    ip = _tc_packT(item_table)
    u_g, i_g = _sc_gather_pair(up, ip, ui >> 2, ii >> 2)
    u_sub = (ui & 3).reshape(_B, 1)
    i_sub = (ii & 3).reshape(_B, 1)
    return _tc_towers(u_g, i_g, u_sub, i_sub,
                      uW0, ub0, uW1, ub1, uW2, ub2,
                      iW0, ib0, iW1, ib1, iW2, ib2)


# speedup vs baseline: 3.6119x; 1.0075x over previous
import functools

import jax
import jax.numpy as jnp
from jax import lax
from jax.experimental import pallas as pl
from jax.experimental.pallas import tpu as pltpu
from jax.experimental.pallas import tpu_sc as plsc

_B = 16384
_E = 32
_NC = 2
_NS = 16
_NW = _NC * _NS
_BPW = _B // _NW
_NCH = 8
_CH = _BPW // _NCH

_TC_BLK = 2048
_LB = 32256
_NPB = 32


def _tc_packT(table):
    xt = table.T

    def body(x_ref, o_ref, tmp_ref):
        tmp_ref[...] = x_ref[...].T
        o_ref[...] = jnp.concatenate(
            [tmp_ref[q::4, :] for q in range(4)], axis=1)

    return pl.pallas_call(
        body,
        grid=(_NPB,),
        in_specs=[pl.BlockSpec((_E, _LB), lambda i: (0, i))],
        out_specs=pl.BlockSpec((_LB // 4, 128), lambda i: (i, 0)),
        out_shape=jax.ShapeDtypeStruct((_NPB * (_LB // 4), 128),
                                       jnp.float32),
        scratch_shapes=[pltpu.VMEM((_LB, _E), jnp.float32)],
    )(xt)


def _sc_gather(packed, hi):
    mesh = plsc.VectorSubcoreMesh(core_axis_name="c", subcore_axis_name="s")

    @functools.partial(
        pl.kernel,
        mesh=mesh,
        out_type=jax.ShapeDtypeStruct((_B, 128), jnp.float32),
        scratch_types=[
            pltpu.VMEM((_BPW,), jnp.int32),
            pltpu.VMEM((_BPW, 128), jnp.float32),
            pltpu.SemaphoreType.DMA,
        ],
    )
    def gather_kernel(tab_hbm, i_hbm, o_hbm, idx_v, rows_v, sem):
        wid = lax.axis_index("s") * _NC + lax.axis_index("c")
        base = wid * _BPW
        pltpu.sync_copy(i_hbm.at[pl.ds(base, _BPW)], idx_v)
        for ch in range(_NCH):
            pltpu.async_copy(
                tab_hbm.at[idx_v.at[pl.ds(ch * _CH, _CH)]],
                rows_v.at[pl.ds(ch * _CH, _CH), :], sem)
        pltpu.make_async_copy(tab_hbm.at[pl.ds(0, _BPW)], rows_v,
                              sem).wait()
        pltpu.sync_copy(rows_v, o_hbm.at[pl.ds(base, _BPW)])

    return gather_kernel(packed, hi)


def _select_sub(g, sub):
    out = jnp.where(sub == 0, g[:, 0 * _E:1 * _E], 0.0)
    out += jnp.where(sub == 1, g[:, 1 * _E:2 * _E], 0.0)
    out += jnp.where(sub == 2, g[:, 2 * _E:3 * _E], 0.0)
    out += jnp.where(sub == 3, g[:, 3 * _E:4 * _E], 0.0)
    return out


def _tower(x, W0, b0, W1, b1, W2, b2):
    def mm(v, W):
        return jnp.dot(v.astype(jnp.bfloat16), W.astype(jnp.bfloat16),
                       preferred_element_type=jnp.float32)
    h = jnp.maximum(mm(x, W0) + b0, 0.0)
    h = jnp.maximum(mm(h, W1) + b1, 0.0)
    return mm(h, W2) + b2


def _tc_body(ug_ref, ig_ref, us_ref, is_ref,
             uW0r, ub0r, uW1r, ub1r, uW2r, ub2r,
             iW0r, ib0r, iW1r, ib1r, iW2r, ib2r, o_ref):
    u = _tower(_select_sub(ug_ref[...], us_ref[...]),
               uW0r[...], ub0r[...], uW1r[...], ub1r[...],
               uW2r[...], ub2r[...])
    v = _tower(_select_sub(ig_ref[...], is_ref[...]),
               iW0r[...], ib0r[...], iW1r[...], ib1r[...],
               iW2r[...], ib2r[...])
    dot = jnp.sum(u * v, axis=1)
    nu = jnp.sqrt(jnp.sum(u * u, axis=1))
    nv = jnp.sqrt(jnp.sum(v * v, axis=1))
    o_ref[...] = dot / (jnp.maximum(nu, 1e-12) * jnp.maximum(nv, 1e-12))


def _tc_towers(u_g, i_g, u_sub, i_sub,
               uW0, ub0, uW1, ub1, uW2, ub2,
               iW0, ib0, iW1, ib1, iW2, ib2):
    def full(a):
        return pl.BlockSpec(a.shape, lambda i: (0,) * a.ndim)

    weights = [uW0, ub0, uW1, ub1, uW2, ub2, iW0, ib0, iW1, ib1, iW2, ib2]
    weights = [w.reshape(1, -1) if w.ndim == 1 else w for w in weights]
    return pl.pallas_call(
        _tc_body,
        grid=(_B // _TC_BLK,),
        in_specs=[
            pl.BlockSpec((_TC_BLK, 128), lambda i: (i, 0)),
            pl.BlockSpec((_TC_BLK, 128), lambda i: (i, 0)),
            pl.BlockSpec((_TC_BLK, 1), lambda i: (i, 0)),
            pl.BlockSpec((_TC_BLK, 1), lambda i: (i, 0)),
        ] + [full(w) for w in weights],
        out_specs=pl.BlockSpec((_TC_BLK,), lambda i: (i,)),
        out_shape=jax.ShapeDtypeStruct((_B,), jnp.float32),
    )(u_g, i_g, u_sub, i_sub, *weights)


@jax.jit
def kernel(user_idx, item_idx, user_table, item_table,
           uW0, ub0, uW1, ub1, uW2, ub2,
           iW0, ib0, iW1, ib1, iW2, ib2):
    ui = user_idx.astype(jnp.int32)
    ii = item_idx.astype(jnp.int32)
    up = _tc_packT(user_table)
    u_g = _sc_gather(up, ui >> 2)
    ip = _tc_packT(item_table)
    i_g = _sc_gather(ip, ii >> 2)
    u_sub = (ui & 3).reshape(_B, 1)
    i_sub = (ii & 3).reshape(_B, 1)
    return _tc_towers(u_g, i_g, u_sub, i_sub,
                      uW0, ub0, uW1, ub1, uW2, ub2,
                      iW0, ib0, iW1, ib1, iW2, ib2)
